# baseline (speedup 1.0000x reference)
import jax
import jax.numpy as jnp
from jax import lax
from jax.experimental import pallas as pl
from jax.experimental.pallas import tpu as pltpu
from jax.experimental.pallas import tpu_sc as plsc

_B, _S, _D = 4, 8192, 768
_N = _S * _D
_NC, _NS = 2, 16
_NW = _NC * _NS
_NPW = _N // _NW
_CH = 16384
_NCH = _NPW // _CH


def _sc_body(x_hbm, pe_hbm, o_hbm, pe_buf, xb, insems, outsems):
    wid = lax.axis_index("s") * _NC + lax.axis_index("c")
    base = wid * _NPW
    for c in range(_NCH):
        off = base + c * _CH
        pltpu.sync_copy(pe_hbm.at[pl.ds(off, _CH)], pe_buf)
        in_cps = []
        for b in range(_B):
            in_cps.append(
                pltpu.async_copy(
                    x_hbm.at[pl.ds(b * _N + off, _CH)], xb.at[b], insems.at[b]
                )
            )
        out_cps = []
        for b in range(_B):
            in_cps[b].wait()

            @plsc.parallel_loop(0, _CH, step=16, unroll=8)
            def _(i, _b=b):
                sl = pl.ds(i, 16)
                xb[_b, sl] = xb[_b, sl] + pe_buf[sl]

            out_cps.append(
                pltpu.async_copy(
                    xb.at[b], o_hbm.at[pl.ds(b * _N + off, _CH)], outsems.at[b]
                )
            )
        for cp in out_cps:
            cp.wait()


def kernel(x, pe_table):
    x_flat = x.reshape(_B * _N)
    pe_flat = pe_table.reshape(_N)
    mesh = plsc.VectorSubcoreMesh(core_axis_name="c", subcore_axis_name="s")
    run = pl.kernel(
        _sc_body,
        out_type=jax.ShapeDtypeStruct((_B * _N,), jnp.float32),
        mesh=mesh,
        scratch_types=[
            pltpu.VMEM((_CH,), jnp.float32),
            pltpu.VMEM((_B, _CH), jnp.float32),
            pltpu.SemaphoreType.DMA((_B,)),
            pltpu.SemaphoreType.DMA((_B,)),
        ],
    )
    out = run(x_flat, pe_flat)
    return out.reshape(_B, _S, _D)


# speedup vs baseline: 2.4491x; 2.4491x over previous
import jax
import jax.numpy as jnp
from jax import lax
from jax.experimental import pallas as pl
from jax.experimental.pallas import tpu as pltpu
from jax.experimental.pallas import tpu_sc as plsc

_B, _S, _D = 4, 8192, 768
_NC, _NS = 2, 16
_NW = _NC * _NS
_RPW = _S // _NW
_CHR = 16
_NCH = _RPW // _CHR


def _sc_body(x_hbm, pe_hbm, o_hbm, pe_buf, xb, insems, outsems):
    wid = lax.axis_index("s") * _NC + lax.axis_index("c")
    row0 = wid * _RPW

    @pl.loop(0, _NCH)
    def _chunk(c):
        r = row0 + c * _CHR
        pltpu.sync_copy(pe_hbm.at[pl.ds(r, _CHR)], pe_buf)
        in_cps = []
        for b in range(_B):
            in_cps.append(
                pltpu.async_copy(
                    x_hbm.at[b, pl.ds(r, _CHR)], xb.at[b], insems.at[b]
                )
            )
        out_cps = []
        for b in range(_B):
            in_cps[b].wait()

            @pl.loop(0, _CHR)
            def _row(rr, _b=b):
                @plsc.parallel_loop(0, _D, step=16, unroll=8)
                def _(j):
                    sl = pl.ds(j, 16)
                    xb[_b, rr, sl] = xb[_b, rr, sl] + pe_buf[rr, sl]

            out_cps.append(
                pltpu.async_copy(
                    xb.at[b], o_hbm.at[b, pl.ds(r, _CHR)], outsems.at[b]
                )
            )
        for cp in out_cps:
            cp.wait()


def kernel(x, pe_table):
    mesh = plsc.VectorSubcoreMesh(core_axis_name="c", subcore_axis_name="s")
    run = pl.kernel(
        _sc_body,
        out_type=jax.ShapeDtypeStruct((_B, _S, _D), jnp.float32),
        mesh=mesh,
        scratch_types=[
            pltpu.VMEM((_CHR, _D), jnp.float32),
            pltpu.VMEM((_B, _CHR, _D), jnp.float32),
            pltpu.SemaphoreType.DMA((_B,)),
            pltpu.SemaphoreType.DMA((_B,)),
        ],
    )
    return run(x, pe_table)
